# baseline (speedup 1.0000x reference)
import functools

import jax
import jax.numpy as jnp
from jax import lax
from jax.experimental import pallas as pl
from jax.experimental.pallas import tpu as pltpu
from jax.experimental.pallas import tpu_sc as plsc

D_FEAT = 128
EMBED = 128
NUM_SAMPLE = 25

NC = 2
NS = 16
NW = NC * NS

PAIRS_PER_OP = 100
BATCH_PER_OP = PAIRS_PER_OP // NUM_SAMPLE
NBUF = 4
COL_CHUNKS = D_FEAT // 16


def _sc_aggregate(table, idx3, b_pad):
  n_ops = idx3.shape[1]
  bpw = b_pad // NW

  mesh = plsc.VectorSubcoreMesh(
      core_axis_name="c", subcore_axis_name="s", num_cores=NC, num_subcores=NS)

  @functools.partial(
      pl.kernel,
      mesh=mesh,
      out_type=jax.ShapeDtypeStruct((b_pad, D_FEAT), jnp.float32),
      scratch_types=[
          pltpu.VMEM((n_ops, PAIRS_PER_OP), jnp.int32),
          pltpu.VMEM((bpw, D_FEAT), jnp.float32),
      ] + [pltpu.VMEM((PAIRS_PER_OP, D_FEAT), jnp.float32) for _ in range(NBUF)]
        + [pltpu.SemaphoreType.DMA for _ in range(NBUF)],
  )
  def agg_kernel(table_hbm, idx_hbm, out_hbm, idx_v, agg_v, *bufs_and_sems):
    bufs = bufs_and_sems[:NBUF]
    sems = bufs_and_sems[NBUF:]
    wid = lax.axis_index("s") * NC + lax.axis_index("c")

    pltpu.sync_copy(idx_hbm.at[wid], idx_v)

    for b in range(NBUF):
      pltpu.async_copy(table_hbm.at[idx_v.at[b]], bufs[b], sems[b])

    def reduce_chunk(j, buf):
      def batch_body(b, _):
        row0 = b * NUM_SAMPLE
        for c in range(COL_CHUNKS):
          cs = c * 16
          acc = buf[row0, pl.ds(cs, 16)]
          for s in range(1, NUM_SAMPLE):
            acc = acc + buf[row0 + s, pl.ds(cs, 16)]
          agg_v[j * BATCH_PER_OP + b, pl.ds(cs, 16)] = acc
        return 0
      lax.fori_loop(0, BATCH_PER_OP, batch_body, 0)

    def outer(jo, _):
      for db in range(NBUF):
        j = jo * NBUF + db
        pltpu.make_async_copy(table_hbm.at[idx_v.at[j]], bufs[db], sems[db]).wait()
        reduce_chunk(j, bufs[db])
        nxt = j + NBUF

        @pl.when(nxt < n_ops)
        def _():
          pltpu.async_copy(table_hbm.at[idx_v.at[nxt]], bufs[db], sems[db])
      return 0

    lax.fori_loop(0, n_ops // NBUF, outer, 0)

    pltpu.sync_copy(agg_v, out_hbm.at[pl.ds(wid * bpw, bpw)])

  return agg_kernel(table, idx3)


def _tc_linear_relu(w, agg, b_pad):
  bblk = 2048
  grid = (b_pad // bblk,)

  def mm_body(w_ref, agg_ref, out_ref):
    out_ref[...] = jnp.maximum(
        lax.dot_general(w_ref[...], agg_ref[...],
                        (((1,), (1,)), ((), ())),
                        preferred_element_type=jnp.float32),
        0.0)

  return pl.pallas_call(
      mm_body,
      grid=grid,
      in_specs=[
          pl.BlockSpec((EMBED, D_FEAT), lambda i: (0, 0)),
          pl.BlockSpec((bblk, D_FEAT), lambda i: (i, 0)),
      ],
      out_specs=pl.BlockSpec((EMBED, bblk), lambda i: (0, i)),
      out_shape=jax.ShapeDtypeStruct((EMBED, b_pad), jnp.float32),
  )(w, agg)


def kernel(nodes, neigh_idx, node_features, W):
  batch = neigh_idx.shape[0]
  b_pad = 10240

  idx_flat = neigh_idx.reshape(-1)
  pad = b_pad * NUM_SAMPLE - idx_flat.shape[0]
  idx_flat = jnp.concatenate([idx_flat, jnp.zeros((pad,), jnp.int32)])
  pairs_per_worker = b_pad * NUM_SAMPLE // NW
  idx3 = idx_flat.reshape(NW, pairs_per_worker // PAIRS_PER_OP, PAIRS_PER_OP)

  agg = _sc_aggregate(node_features, idx3, b_pad)
  out = _tc_linear_relu(W * (1.0 / NUM_SAMPLE), agg, b_pad)
  return out[:, :batch]


# speedup vs baseline: 2.6213x; 2.6213x over previous
import functools

import jax
import jax.numpy as jnp
from jax import lax
from jax.experimental import pallas as pl
from jax.experimental.pallas import tpu as pltpu
from jax.experimental.pallas import tpu_sc as plsc

D_FEAT = 128
EMBED = 128
NUM_SAMPLE = 25

NC = 2
NS = 16
NW = NC * NS

PAIRS_PER_OP = 100
BATCH_PER_OP = PAIRS_PER_OP // NUM_SAMPLE
NBUF = 2
COL_CHUNKS = D_FEAT // 16


def _sc_aggregate(table, idx3, b_pad):
  n_ops = idx3.shape[1]
  bpw = b_pad // NW
  n_nodes = table.shape[0]
  rows_per_stage = n_nodes // NS

  mesh = plsc.VectorSubcoreMesh(
      core_axis_name="c", subcore_axis_name="s", num_cores=NC, num_subcores=NS)

  @functools.partial(
      pl.kernel,
      mesh=mesh,
      out_type=jax.ShapeDtypeStruct((b_pad * D_FEAT,), jnp.float32),
      scratch_types=[
          pltpu.VMEM((n_ops, PAIRS_PER_OP), jnp.int32),
          pltpu.MemorySpace.VMEM_SHARED((n_nodes, D_FEAT), jnp.float32),
      ] + [pltpu.VMEM((PAIRS_PER_OP, D_FEAT), jnp.float32) for _ in range(NBUF)]
        + [pltpu.VMEM((BATCH_PER_OP * D_FEAT,), jnp.float32) for _ in range(NBUF)]
        + [pltpu.SemaphoreType.DMA for _ in range(2 * NBUF)],
  )
  def agg_kernel(table_hbm, idx_hbm, out_hbm, idx_v, table_sh,
                 *bufs_and_sems):
    bufs = bufs_and_sems[:NBUF]
    obufs = bufs_and_sems[NBUF:2 * NBUF]
    sems = bufs_and_sems[2 * NBUF:3 * NBUF]
    osems = bufs_and_sems[3 * NBUF:]
    sid = lax.axis_index("s")
    wid = sid * NC + lax.axis_index("c")
    obase = wid * bpw * D_FEAT

    r0 = sid * rows_per_stage
    pltpu.sync_copy(table_hbm.at[pl.ds(r0, rows_per_stage)],
                    table_sh.at[pl.ds(r0, rows_per_stage)])
    pltpu.sync_copy(idx_hbm.at[wid], idx_v)
    plsc.subcore_barrier()

    for b in range(NBUF):
      pltpu.async_copy(table_sh.at[idx_v.at[b]], bufs[b], sems[b])

    def reduce_chunk(buf, obuf):
      def batch_body(b, _):
        row0 = b * NUM_SAMPLE
        ob = pl.multiple_of(b * D_FEAT, D_FEAT)
        for c in range(COL_CHUNKS):
          cs = c * 16
          acc = buf[row0, pl.ds(cs, 16)]
          for s in range(1, NUM_SAMPLE):
            acc = acc + buf[row0 + s, pl.ds(cs, 16)]
          obuf[pl.ds(ob + cs, 16)] = acc
        return 0
      lax.fori_loop(0, BATCH_PER_OP, batch_body, 0)

    def out_slice(j):
      return out_hbm.at[pl.ds(obase + j * BATCH_PER_OP * D_FEAT,
                              BATCH_PER_OP * D_FEAT)]

    def outer(jo, _):
      for db in range(NBUF):
        j = jo * NBUF + db
        pltpu.make_async_copy(table_sh.at[idx_v.at[j]], bufs[db], sems[db]).wait()

        @pl.when(j >= NBUF)
        def _():
          pltpu.make_async_copy(obufs[db], out_slice(j), osems[db]).wait()

        reduce_chunk(bufs[db], obufs[db])
        pltpu.async_copy(obufs[db], out_slice(j), osems[db])
        nxt = j + NBUF

        @pl.when(nxt < n_ops)
        def _():
          pltpu.async_copy(table_sh.at[idx_v.at[nxt]], bufs[db], sems[db])
      return 0

    lax.fori_loop(0, n_ops // NBUF, outer, 0)

    for db in range(NBUF):
      pltpu.make_async_copy(obufs[db], out_slice(0), osems[db]).wait()

  return agg_kernel(table, idx3)


def _tc_linear_relu(w, agg, b_pad):
  bblk = 2048
  grid = (b_pad // bblk,)

  def mm_body(w_ref, agg_ref, out_ref):
    out_ref[...] = jnp.maximum(
        lax.dot_general(w_ref[...], agg_ref[...],
                        (((1,), (1,)), ((), ())),
                        preferred_element_type=jnp.float32),
        0.0)

  return pl.pallas_call(
      mm_body,
      grid=grid,
      in_specs=[
          pl.BlockSpec((EMBED, D_FEAT), lambda i: (0, 0)),
          pl.BlockSpec((bblk, D_FEAT), lambda i: (i, 0)),
      ],
      out_specs=pl.BlockSpec((EMBED, bblk), lambda i: (0, i)),
      out_shape=jax.ShapeDtypeStruct((EMBED, b_pad), jnp.float32),
  )(w, agg)


def kernel(nodes, neigh_idx, node_features, W):
  batch = neigh_idx.shape[0]
  b_pad = 10240

  idx_flat = neigh_idx.reshape(-1)
  pad = b_pad * NUM_SAMPLE - idx_flat.shape[0]
  idx_flat = jnp.concatenate([idx_flat, jnp.zeros((pad,), jnp.int32)])
  pairs_per_worker = b_pad * NUM_SAMPLE // NW
  idx3 = idx_flat.reshape(NW, pairs_per_worker // PAIRS_PER_OP, PAIRS_PER_OP)

  n_nodes = node_features.shape[0]
  n_pad = ((n_nodes + 8 * NS - 1) // (8 * NS)) * (8 * NS)
  table = jnp.pad(node_features, ((0, n_pad - n_nodes), (0, 0)))

  agg = _sc_aggregate(table, idx3, b_pad).reshape(b_pad, D_FEAT)
  out = _tc_linear_relu(W * (1.0 / NUM_SAMPLE), agg, b_pad)
  return out[:, :batch]


# speedup vs baseline: 2.6857x; 1.0246x over previous
import functools

import jax
import jax.numpy as jnp
from jax import lax
from jax.experimental import pallas as pl
from jax.experimental.pallas import tpu as pltpu
from jax.experimental.pallas import tpu_sc as plsc

D_FEAT = 128
EMBED = 128
NUM_SAMPLE = 25

NC = 2
NS = 16
NW = NC * NS

PAIRS_PER_OP = 100
BATCH_PER_OP = PAIRS_PER_OP // NUM_SAMPLE
NBUF = 2
COL_CHUNKS = D_FEAT // 16


def _sc_aggregate(table, idx3, b_pad):
  n_ops = idx3.shape[1]
  bpw = b_pad // NW
  n_nodes = table.shape[0]
  stage_workers = next(k for k in range(NS, 0, -1)
                       if n_nodes % k == 0 and (n_nodes // k) % 8 == 0)
  rows_per_stage = n_nodes // stage_workers

  mesh = plsc.VectorSubcoreMesh(
      core_axis_name="c", subcore_axis_name="s", num_cores=NC, num_subcores=NS)

  @functools.partial(
      pl.kernel,
      mesh=mesh,
      out_type=jax.ShapeDtypeStruct((b_pad * D_FEAT,), jnp.float32),
      scratch_types=[
          pltpu.VMEM((n_ops, PAIRS_PER_OP), jnp.int32),
          pltpu.MemorySpace.VMEM_SHARED((n_nodes, D_FEAT), jnp.float32),
      ] + [pltpu.VMEM((PAIRS_PER_OP, D_FEAT), jnp.float32) for _ in range(NBUF)]
        + [pltpu.VMEM((BATCH_PER_OP * D_FEAT,), jnp.float32) for _ in range(NBUF)]
        + [pltpu.SemaphoreType.DMA for _ in range(2 * NBUF)],
  )
  def agg_kernel(table_hbm, idx_hbm, out_hbm, idx_v, table_sh,
                 *bufs_and_sems):
    bufs = bufs_and_sems[:NBUF]
    obufs = bufs_and_sems[NBUF:2 * NBUF]
    sems = bufs_and_sems[2 * NBUF:3 * NBUF]
    osems = bufs_and_sems[3 * NBUF:]
    sid = lax.axis_index("s")
    wid = sid * NC + lax.axis_index("c")
    obase = wid * bpw * D_FEAT

    @pl.when(sid < stage_workers)
    def _():
      r0 = pl.multiple_of(sid * rows_per_stage, 8)
      pltpu.sync_copy(table_hbm.at[pl.ds(r0, rows_per_stage)],
                      table_sh.at[pl.ds(r0, rows_per_stage)])

    pltpu.sync_copy(idx_hbm.at[wid], idx_v)
    plsc.subcore_barrier()

    for b in range(NBUF):
      pltpu.async_copy(table_sh.at[idx_v.at[b]], bufs[b], sems[b])

    def reduce_chunk(buf, obuf):
      def batch_body(b, _):
        row0 = b * NUM_SAMPLE
        ob = pl.multiple_of(b * D_FEAT, D_FEAT)
        for c in range(COL_CHUNKS):
          cs = c * 16
          acc = buf[row0, pl.ds(cs, 16)]
          for s in range(1, NUM_SAMPLE):
            acc = acc + buf[row0 + s, pl.ds(cs, 16)]
          obuf[pl.ds(ob + cs, 16)] = acc
        return 0
      lax.fori_loop(0, BATCH_PER_OP, batch_body, 0)

    def out_slice(j):
      return out_hbm.at[pl.ds(obase + j * BATCH_PER_OP * D_FEAT,
                              BATCH_PER_OP * D_FEAT)]

    def outer(jo, _):
      for db in range(NBUF):
        j = jo * NBUF + db
        pltpu.make_async_copy(table_sh.at[idx_v.at[j]], bufs[db], sems[db]).wait()

        @pl.when(j >= NBUF)
        def _():
          pltpu.make_async_copy(obufs[db], out_slice(j), osems[db]).wait()

        reduce_chunk(bufs[db], obufs[db])
        pltpu.async_copy(obufs[db], out_slice(j), osems[db])
        nxt = j + NBUF

        @pl.when(nxt < n_ops)
        def _():
          pltpu.async_copy(table_sh.at[idx_v.at[nxt]], bufs[db], sems[db])
      return 0

    lax.fori_loop(0, n_ops // NBUF, outer, 0)

    for db in range(NBUF):
      pltpu.make_async_copy(obufs[db], out_slice(0), osems[db]).wait()

  return agg_kernel(table, idx3)


def _tc_linear_relu(w, agg, b_pad):
  bblk = 2048
  grid = (b_pad // bblk,)

  def mm_body(w_ref, agg_ref, out_ref):
    out_ref[...] = jnp.maximum(
        lax.dot_general(w_ref[...], agg_ref[...],
                        (((1,), (1,)), ((), ())),
                        preferred_element_type=jnp.float32),
        0.0)

  return pl.pallas_call(
      mm_body,
      grid=grid,
      in_specs=[
          pl.BlockSpec((EMBED, D_FEAT), lambda i: (0, 0)),
          pl.BlockSpec((bblk, D_FEAT), lambda i: (i, 0)),
      ],
      out_specs=pl.BlockSpec((EMBED, bblk), lambda i: (0, i)),
      out_shape=jax.ShapeDtypeStruct((EMBED, b_pad), jnp.float32),
  )(w, agg)


def kernel(nodes, neigh_idx, node_features, W):
  batch = neigh_idx.shape[0]
  b_pad = 10240

  idx_flat = neigh_idx.reshape(-1)
  pad = b_pad * NUM_SAMPLE - idx_flat.shape[0]
  idx_flat = jnp.concatenate([idx_flat, jnp.zeros((pad,), jnp.int32)])
  pairs_per_worker = b_pad * NUM_SAMPLE // NW
  idx3 = idx_flat.reshape(NW, pairs_per_worker // PAIRS_PER_OP, PAIRS_PER_OP)

  agg = _sc_aggregate(node_features, idx3, b_pad).reshape(b_pad, D_FEAT)
  out = _tc_linear_relu(W * (1.0 / NUM_SAMPLE), agg, b_pad)
  return out[:, :batch]


# speedup vs baseline: 2.6990x; 1.0049x over previous
import functools

import jax
import jax.numpy as jnp
from jax import lax
from jax.experimental import pallas as pl
from jax.experimental.pallas import tpu as pltpu
from jax.experimental.pallas import tpu_sc as plsc

D_FEAT = 128
EMBED = 128
NUM_SAMPLE = 25

NC = 2
NS = 16
NW = NC * NS

PAIRS_PER_OP = 100
BATCH_PER_OP = PAIRS_PER_OP // NUM_SAMPLE
NBUF = 2
COL_CHUNKS = D_FEAT // 16


def _sc_aggregate(table, idx3, b_pad):
  n_ops = idx3.shape[1]
  bpw = b_pad // NW
  n_nodes = table.shape[0]
  stage_workers = next(k for k in range(NS, 0, -1)
                       if n_nodes % k == 0 and (n_nodes // k) % 8 == 0)
  rows_per_stage = n_nodes // stage_workers

  mesh = plsc.VectorSubcoreMesh(
      core_axis_name="c", subcore_axis_name="s", num_cores=NC, num_subcores=NS)

  @functools.partial(
      pl.kernel,
      mesh=mesh,
      out_type=jax.ShapeDtypeStruct((b_pad * D_FEAT,), jnp.float32),
      scratch_types=[
          pltpu.VMEM((n_ops, PAIRS_PER_OP), jnp.int32),
          pltpu.MemorySpace.VMEM_SHARED((n_nodes, D_FEAT), jnp.float32),
      ] + [pltpu.VMEM((PAIRS_PER_OP, D_FEAT), jnp.float32) for _ in range(NBUF)]
        + [pltpu.VMEM((BATCH_PER_OP * D_FEAT,), jnp.float32) for _ in range(NBUF)]
        + [pltpu.SemaphoreType.DMA for _ in range(2 * NBUF)],
  )
  def agg_kernel(table_hbm, idx_hbm, out_hbm, idx_v, table_sh,
                 *bufs_and_sems):
    bufs = bufs_and_sems[:NBUF]
    obufs = bufs_and_sems[NBUF:2 * NBUF]
    sems = bufs_and_sems[2 * NBUF:3 * NBUF]
    osems = bufs_and_sems[3 * NBUF:]
    sid = lax.axis_index("s")
    wid = sid * NC + lax.axis_index("c")
    obase = wid * bpw * D_FEAT

    @pl.when(sid < stage_workers)
    def _():
      r0 = pl.multiple_of(sid * rows_per_stage, 8)
      pltpu.sync_copy(table_hbm.at[pl.ds(r0, rows_per_stage)],
                      table_sh.at[pl.ds(r0, rows_per_stage)])

    pltpu.sync_copy(idx_hbm.at[wid], idx_v)
    plsc.subcore_barrier()

    for b in range(NBUF):
      pltpu.async_copy(table_sh.at[idx_v.at[b]], bufs[b], sems[b])

    def reduce_chunk(buf, obuf):
      def batch_body(b, _):
        row0 = b * NUM_SAMPLE
        ob = pl.multiple_of(b * D_FEAT, D_FEAT)
        for c in range(COL_CHUNKS):
          cs = c * 16
          acc = buf[row0, pl.ds(cs, 16)]
          for s in range(1, NUM_SAMPLE):
            acc = acc + buf[row0 + s, pl.ds(cs, 16)]
          obuf[pl.ds(ob + cs, 16)] = acc
        return 0
      lax.fori_loop(0, BATCH_PER_OP, batch_body, 0)

    def out_slice(j):
      return out_hbm.at[pl.ds(obase + j * BATCH_PER_OP * D_FEAT,
                              BATCH_PER_OP * D_FEAT)]

    def outer(jo, _):
      for db in range(NBUF):
        j = jo * NBUF + db
        pltpu.make_async_copy(table_sh.at[idx_v.at[j]], bufs[db], sems[db]).wait()

        @pl.when(j >= NBUF)
        def _():
          pltpu.make_async_copy(obufs[db], out_slice(j), osems[db]).wait()

        reduce_chunk(bufs[db], obufs[db])
        pltpu.async_copy(obufs[db], out_slice(j), osems[db])
        nxt = j + NBUF

        @pl.when(nxt < n_ops)
        def _():
          pltpu.async_copy(table_sh.at[idx_v.at[nxt]], bufs[db], sems[db])
      return 0

    lax.fori_loop(0, n_ops // NBUF, outer, 0)

    for db in range(NBUF):
      pltpu.make_async_copy(obufs[db], out_slice(0), osems[db]).wait()

  return agg_kernel(table, idx3)


def _tc_linear_relu(w, agg, batch):
  def mm_body(w_ref, agg_ref, out_ref):
    out_ref[...] = jnp.maximum(
        lax.dot_general(w_ref[...], agg_ref[...],
                        (((1,), (1,)), ((), ())),
                        preferred_element_type=jnp.float32),
        0.0)

  return pl.pallas_call(
      mm_body,
      grid=(1,),
      in_specs=[
          pl.BlockSpec((EMBED, D_FEAT), lambda i: (0, 0)),
          pl.BlockSpec((batch, D_FEAT), lambda i: (0, 0)),
      ],
      out_specs=pl.BlockSpec((EMBED, batch), lambda i: (0, 0)),
      out_shape=jax.ShapeDtypeStruct((EMBED, batch), jnp.float32),
  )(w, agg)


def kernel(nodes, neigh_idx, node_features, W):
  batch = neigh_idx.shape[0]
  b_pad = 10240

  idx_flat = neigh_idx.reshape(-1)
  pad = b_pad * NUM_SAMPLE - idx_flat.shape[0]
  idx_flat = jnp.concatenate([idx_flat, jnp.zeros((pad,), jnp.int32)])
  pairs_per_worker = b_pad * NUM_SAMPLE // NW
  idx3 = idx_flat.reshape(NW, pairs_per_worker // PAIRS_PER_OP, PAIRS_PER_OP)

  agg = _sc_aggregate(node_features, idx3, b_pad).reshape(b_pad, D_FEAT)
  return _tc_linear_relu(W * (1.0 / NUM_SAMPLE), agg, batch)


# speedup vs baseline: 4.1841x; 1.5503x over previous
import functools

import jax
import jax.numpy as jnp
from jax import lax
from jax.experimental import pallas as pl
from jax.experimental.pallas import tpu as pltpu
from jax.experimental.pallas import tpu_sc as plsc

D_FEAT = 128
EMBED = 128
NUM_SAMPLE = 25

NC = 2
NS = 16
NW = NC * NS

PAIRS_PER_OP = 50
BATCH_PER_OP = PAIRS_PER_OP // NUM_SAMPLE
NBUF = 4
COL_CHUNKS = D_FEAT // 16


def _sc_aggregate(table, idx3, b_pad):
  n_ops = idx3.shape[1]
  bpw = b_pad // NW
  n_nodes = table.shape[0]
  stage_workers = next(k for k in range(NS, 0, -1)
                       if n_nodes % k == 0 and (n_nodes // k) % 8 == 0)
  rows_per_stage = n_nodes // stage_workers

  mesh = plsc.VectorSubcoreMesh(
      core_axis_name="c", subcore_axis_name="s", num_cores=NC, num_subcores=NS)

  @functools.partial(
      pl.kernel,
      mesh=mesh,
      out_type=jax.ShapeDtypeStruct((b_pad * D_FEAT,), jnp.float32),
      scratch_types=[
          pltpu.VMEM((n_ops, PAIRS_PER_OP), jnp.int32),
          pltpu.MemorySpace.VMEM_SHARED((n_nodes, D_FEAT), jnp.float32),
      ] + [pltpu.VMEM((PAIRS_PER_OP, D_FEAT), jnp.float32) for _ in range(NBUF)]
        + [pltpu.VMEM((BATCH_PER_OP * D_FEAT,), jnp.float32) for _ in range(NBUF)]
        + [pltpu.SemaphoreType.DMA for _ in range(2 * NBUF)],
  )
  def agg_kernel(table_hbm, idx_hbm, out_hbm, idx_v, table_sh,
                 *bufs_and_sems):
    bufs = bufs_and_sems[:NBUF]
    obufs = bufs_and_sems[NBUF:2 * NBUF]
    sems = bufs_and_sems[2 * NBUF:3 * NBUF]
    osems = bufs_and_sems[3 * NBUF:]
    sid = lax.axis_index("s")
    wid = sid * NC + lax.axis_index("c")
    obase = wid * bpw * D_FEAT

    @pl.when(sid < stage_workers)
    def _():
      r0 = pl.multiple_of(sid * rows_per_stage, 8)
      pltpu.sync_copy(table_hbm.at[pl.ds(r0, rows_per_stage)],
                      table_sh.at[pl.ds(r0, rows_per_stage)])

    pltpu.sync_copy(idx_hbm.at[wid], idx_v)
    plsc.subcore_barrier()

    for b in range(NBUF):
      pltpu.async_copy(table_sh.at[idx_v.at[b]], bufs[b], sems[b])

    def reduce_chunk(buf, obuf):
      def batch_body(b, _):
        row0 = b * NUM_SAMPLE
        ob = pl.multiple_of(b * D_FEAT, D_FEAT)
        for c in range(COL_CHUNKS):
          cs = c * 16
          accs = [buf[row0 + k, pl.ds(cs, 16)] for k in range(2)]
          for s in range(2, NUM_SAMPLE):
            accs[s % 2] = accs[s % 2] + buf[row0 + s, pl.ds(cs, 16)]
          obuf[pl.ds(ob + cs, 16)] = accs[0] + accs[1]
        return 0
      lax.fori_loop(0, BATCH_PER_OP, batch_body, 0)

    def out_slice(j):
      off = pl.multiple_of(obase + j * BATCH_PER_OP * D_FEAT, 8)
      return out_hbm.at[pl.ds(off, BATCH_PER_OP * D_FEAT)]

    def outer(jo, _):
      for db in range(NBUF):
        j = jo * NBUF + db
        pltpu.make_async_copy(table_sh.at[idx_v.at[j]], bufs[db], sems[db]).wait()

        @pl.when(j >= NBUF)
        def _():
          pltpu.make_async_copy(obufs[db], out_slice(j), osems[db]).wait()

        reduce_chunk(bufs[db], obufs[db])
        pltpu.async_copy(obufs[db], out_slice(j), osems[db])
        nxt = j + NBUF

        @pl.when(nxt < n_ops)
        def _():
          pltpu.async_copy(table_sh.at[idx_v.at[nxt]], bufs[db], sems[db])
      return 0

    lax.fori_loop(0, n_ops // NBUF, outer, 0)

    for db in range(NBUF):
      pltpu.make_async_copy(obufs[db], out_slice(0), osems[db]).wait()

  return agg_kernel(table, idx3)


def _tc_linear_relu(w, agg, batch):
  def mm_body(w_ref, agg_ref, out_ref):
    out_ref[...] = jnp.maximum(
        lax.dot_general(w_ref[...], agg_ref[...],
                        (((1,), (1,)), ((), ())),
                        preferred_element_type=jnp.float32),
        0.0)

  return pl.pallas_call(
      mm_body,
      grid=(1,),
      in_specs=[
          pl.BlockSpec((EMBED, D_FEAT), lambda i: (0, 0)),
          pl.BlockSpec((batch, D_FEAT), lambda i: (0, 0)),
      ],
      out_specs=pl.BlockSpec((EMBED, batch), lambda i: (0, 0)),
      out_shape=jax.ShapeDtypeStruct((EMBED, batch), jnp.float32),
  )(w, agg)


def kernel(nodes, neigh_idx, node_features, W):
  batch = neigh_idx.shape[0]
  b_pad = 10240

  idx_flat = neigh_idx.reshape(-1)
  pad = b_pad * NUM_SAMPLE - idx_flat.shape[0]
  idx_flat = jnp.concatenate([idx_flat, jnp.zeros((pad,), jnp.int32)])
  pairs_per_worker = b_pad * NUM_SAMPLE // NW
  idx3 = idx_flat.reshape(NW, pairs_per_worker // PAIRS_PER_OP, PAIRS_PER_OP)

  agg = _sc_aggregate(node_features, idx3, b_pad).reshape(b_pad, D_FEAT)
  return _tc_linear_relu(W * (1.0 / NUM_SAMPLE), agg, batch)


# speedup vs baseline: 4.1920x; 1.0019x over previous
import functools

import jax
import jax.numpy as jnp
from jax import lax
from jax.experimental import pallas as pl
from jax.experimental.pallas import tpu as pltpu
from jax.experimental.pallas import tpu_sc as plsc

D_FEAT = 128
EMBED = 128
NUM_SAMPLE = 25

NC = 2
NS = 16
NW = NC * NS

PAIRS_PER_OP = 50
BATCH_PER_OP = PAIRS_PER_OP // NUM_SAMPLE
NBUF = 4
NOBUF = 2
COL_CHUNKS = D_FEAT // 16


def _sc_aggregate(table, idx3, b_pad):
  n_ops = idx3.shape[1]
  bpw = b_pad // NW
  n_nodes = table.shape[0]
  stage_workers = next(k for k in range(NS, 0, -1)
                       if n_nodes % k == 0 and (n_nodes // k) % 8 == 0)
  rows_per_stage = n_nodes // stage_workers

  mesh = plsc.VectorSubcoreMesh(
      core_axis_name="c", subcore_axis_name="s", num_cores=NC, num_subcores=NS)

  @functools.partial(
      pl.kernel,
      mesh=mesh,
      out_type=jax.ShapeDtypeStruct((b_pad * D_FEAT,), jnp.float32),
      scratch_types=[
          pltpu.VMEM((n_ops, PAIRS_PER_OP), jnp.int32),
          pltpu.MemorySpace.VMEM_SHARED((n_nodes, D_FEAT), jnp.float32),
      ] + [pltpu.VMEM((PAIRS_PER_OP, D_FEAT), jnp.float32) for _ in range(NBUF)]
        + [pltpu.VMEM((BATCH_PER_OP * D_FEAT,), jnp.float32) for _ in range(NOBUF)]
        + [pltpu.SemaphoreType.DMA for _ in range(NBUF + NOBUF)],
  )
  def agg_kernel(table_hbm, idx_hbm, out_hbm, idx_v, table_sh,
                 *bufs_and_sems):
    bufs = bufs_and_sems[:NBUF]
    obufs = bufs_and_sems[NBUF:NBUF + NOBUF]
    sems = bufs_and_sems[NBUF + NOBUF:2 * NBUF + NOBUF]
    osems = bufs_and_sems[2 * NBUF + NOBUF:]
    sid = lax.axis_index("s")
    wid = sid * NC + lax.axis_index("c")
    obase = wid * bpw * D_FEAT

    @pl.when(sid < stage_workers)
    def _():
      r0 = pl.multiple_of(sid * rows_per_stage, 8)
      pltpu.sync_copy(table_hbm.at[pl.ds(r0, rows_per_stage)],
                      table_sh.at[pl.ds(r0, rows_per_stage)])

    pltpu.sync_copy(idx_hbm.at[wid], idx_v)
    plsc.subcore_barrier()

    for b in range(NBUF):
      pltpu.async_copy(table_sh.at[idx_v.at[b]], bufs[b], sems[b])

    def reduce_chunk(buf, obuf):
      def batch_body(b, _):
        row0 = b * NUM_SAMPLE
        ob = pl.multiple_of(b * D_FEAT, D_FEAT)
        for c in range(COL_CHUNKS):
          cs = c * 16
          accs = [buf[row0 + k, pl.ds(cs, 16)] for k in range(2)]
          for s in range(2, NUM_SAMPLE):
            accs[s % 2] = accs[s % 2] + buf[row0 + s, pl.ds(cs, 16)]
          obuf[pl.ds(ob + cs, 16)] = accs[0] + accs[1]
        return 0
      lax.fori_loop(0, BATCH_PER_OP, batch_body, 0)

    def out_slice(j):
      off = pl.multiple_of(obase + j * BATCH_PER_OP * D_FEAT, 8)
      return out_hbm.at[pl.ds(off, BATCH_PER_OP * D_FEAT)]

    def outer(jo, _):
      for db in range(NBUF):
        j = jo * NBUF + db
        ob = db % NOBUF
        pltpu.make_async_copy(table_sh.at[idx_v.at[j]], bufs[db], sems[db]).wait()

        @pl.when(j >= NOBUF)
        def _():
          pltpu.make_async_copy(obufs[ob], out_slice(j), osems[ob]).wait()

        reduce_chunk(bufs[db], obufs[ob])
        pltpu.async_copy(obufs[ob], out_slice(j), osems[ob])
        nxt = j + NBUF

        @pl.when(nxt < n_ops)
        def _():
          pltpu.async_copy(table_sh.at[idx_v.at[nxt]], bufs[db], sems[db])
      return 0

    lax.fori_loop(0, n_ops // NBUF, outer, 0)

    for db in range(NOBUF):
      pltpu.make_async_copy(obufs[db], out_slice(0), osems[db]).wait()

  return agg_kernel(table, idx3)


def _tc_linear_relu(w, agg, batch):
  def mm_body(w_ref, agg_ref, out_ref):
    out_ref[...] = jnp.maximum(
        lax.dot_general(w_ref[...], agg_ref[...],
                        (((1,), (1,)), ((), ())),
                        preferred_element_type=jnp.float32),
        0.0)

  return pl.pallas_call(
      mm_body,
      grid=(1,),
      in_specs=[
          pl.BlockSpec((EMBED, D_FEAT), lambda i: (0, 0)),
          pl.BlockSpec((batch, D_FEAT), lambda i: (0, 0)),
      ],
      out_specs=pl.BlockSpec((EMBED, batch), lambda i: (0, 0)),
      out_shape=jax.ShapeDtypeStruct((EMBED, batch), jnp.float32),
  )(w, agg)


def kernel(nodes, neigh_idx, node_features, W):
  batch = neigh_idx.shape[0]
  b_pad = 10240

  idx_flat = neigh_idx.reshape(-1)
  pad = b_pad * NUM_SAMPLE - idx_flat.shape[0]
  idx_flat = jnp.concatenate([idx_flat, jnp.zeros((pad,), jnp.int32)])
  pairs_per_worker = b_pad * NUM_SAMPLE // NW
  idx3 = idx_flat.reshape(NW, pairs_per_worker // PAIRS_PER_OP, PAIRS_PER_OP)

  agg = _sc_aggregate(node_features, idx3, b_pad).reshape(b_pad, D_FEAT)
  return _tc_linear_relu(W * (1.0 / NUM_SAMPLE), agg, batch)


# speedup vs baseline: 4.1997x; 1.0018x over previous
import functools

import jax
import jax.numpy as jnp
from jax import lax
from jax.experimental import pallas as pl
from jax.experimental.pallas import tpu as pltpu
from jax.experimental.pallas import tpu_sc as plsc

D_FEAT = 128
EMBED = 128
NUM_SAMPLE = 25

NC = 2
NS = 16
NW = NC * NS

PAIRS_PER_OP = 50
BATCH_PER_OP = PAIRS_PER_OP // NUM_SAMPLE
NBUF = 4
NOBUF = 2
COL_CHUNKS = D_FEAT // 16


def _sc_aggregate(table, idx3, b_pad):
  n_ops = idx3.shape[1]
  bpw = b_pad // NW
  n_nodes = table.shape[0]
  stage_workers = next(k for k in range(NS, 0, -1)
                       if n_nodes % k == 0 and (n_nodes // k) % 8 == 0)
  rows_per_stage = n_nodes // stage_workers

  mesh = plsc.VectorSubcoreMesh(
      core_axis_name="c", subcore_axis_name="s", num_cores=NC, num_subcores=NS)

  @functools.partial(
      pl.kernel,
      mesh=mesh,
      out_type=jax.ShapeDtypeStruct((b_pad * D_FEAT,), jnp.float32),
      scratch_types=[
          pltpu.VMEM((n_ops, PAIRS_PER_OP), jnp.int32),
          pltpu.MemorySpace.VMEM_SHARED((n_nodes, D_FEAT), jnp.float32),
      ] + [pltpu.VMEM((PAIRS_PER_OP, D_FEAT), jnp.float32) for _ in range(NBUF)]
        + [pltpu.VMEM((BATCH_PER_OP * D_FEAT,), jnp.float32) for _ in range(NOBUF)]
        + [pltpu.SemaphoreType.DMA for _ in range(NBUF + NOBUF)],
  )
  def agg_kernel(table_hbm, idx_hbm, out_hbm, idx_v, table_sh,
                 *bufs_and_sems):
    bufs = bufs_and_sems[:NBUF]
    obufs = bufs_and_sems[NBUF:NBUF + NOBUF]
    sems = bufs_and_sems[NBUF + NOBUF:2 * NBUF + NOBUF]
    osems = bufs_and_sems[2 * NBUF + NOBUF:]
    sid = lax.axis_index("s")
    wid = sid * NC + lax.axis_index("c")
    obase = wid * bpw * D_FEAT

    @pl.when(sid < stage_workers)
    def _():
      r0 = pl.multiple_of(sid * rows_per_stage, 8)
      pltpu.sync_copy(table_hbm.at[pl.ds(r0, rows_per_stage)],
                      table_sh.at[pl.ds(r0, rows_per_stage)])

    pltpu.sync_copy(idx_hbm.at[wid], idx_v)
    plsc.subcore_barrier()

    for b in range(NBUF):
      pltpu.async_copy(table_sh.at[idx_v.at[b]], bufs[b], sems[b])

    def reduce_chunk(buf, obuf):
      def batch_body(b, _):
        row0 = b * NUM_SAMPLE
        ob = pl.multiple_of(b * D_FEAT, D_FEAT)
        for c in range(COL_CHUNKS):
          cs = c * 16
          accs = [buf[row0 + k, pl.ds(cs, 16)] for k in range(4)]
          for s in range(4, NUM_SAMPLE):
            accs[s % 4] = accs[s % 4] + buf[row0 + s, pl.ds(cs, 16)]
          obuf[pl.ds(ob + cs, 16)] = (accs[0] + accs[1]) + (accs[2] + accs[3])
        return 0
      lax.fori_loop(0, BATCH_PER_OP, batch_body, 0)

    def out_slice(j):
      off = pl.multiple_of(obase + j * BATCH_PER_OP * D_FEAT, 8)
      return out_hbm.at[pl.ds(off, BATCH_PER_OP * D_FEAT)]

    def outer(jo, _):
      for db in range(NBUF):
        j = jo * NBUF + db
        ob = db % NOBUF
        pltpu.make_async_copy(table_sh.at[idx_v.at[j]], bufs[db], sems[db]).wait()

        @pl.when(j >= NOBUF)
        def _():
          pltpu.make_async_copy(obufs[ob], out_slice(j), osems[ob]).wait()

        reduce_chunk(bufs[db], obufs[ob])
        pltpu.async_copy(obufs[ob], out_slice(j), osems[ob])
        nxt = j + NBUF

        @pl.when(nxt < n_ops)
        def _():
          pltpu.async_copy(table_sh.at[idx_v.at[nxt]], bufs[db], sems[db])
      return 0

    lax.fori_loop(0, n_ops // NBUF, outer, 0)

    for db in range(NOBUF):
      pltpu.make_async_copy(obufs[db], out_slice(0), osems[db]).wait()

  return agg_kernel(table, idx3)


def _tc_linear_relu(w, agg, batch):
  def mm_body(w_ref, agg_ref, out_ref):
    out_ref[...] = jnp.maximum(
        lax.dot_general(w_ref[...], agg_ref[...],
                        (((1,), (1,)), ((), ())),
                        preferred_element_type=jnp.float32),
        0.0)

  return pl.pallas_call(
      mm_body,
      grid=(1,),
      in_specs=[
          pl.BlockSpec((EMBED, D_FEAT), lambda i: (0, 0)),
          pl.BlockSpec((batch, D_FEAT), lambda i: (0, 0)),
      ],
      out_specs=pl.BlockSpec((EMBED, batch), lambda i: (0, 0)),
      out_shape=jax.ShapeDtypeStruct((EMBED, batch), jnp.float32),
  )(w, agg)


def kernel(nodes, neigh_idx, node_features, W):
  batch = neigh_idx.shape[0]
  b_pad = 10240

  idx_flat = neigh_idx.reshape(-1)
  pad = b_pad * NUM_SAMPLE - idx_flat.shape[0]
  idx_flat = jnp.concatenate([idx_flat, jnp.zeros((pad,), jnp.int32)])
  pairs_per_worker = b_pad * NUM_SAMPLE // NW
  idx3 = idx_flat.reshape(NW, pairs_per_worker // PAIRS_PER_OP, PAIRS_PER_OP)

  agg = _sc_aggregate(node_features, idx3, b_pad).reshape(b_pad, D_FEAT)
  return _tc_linear_relu(W * (1.0 / NUM_SAMPLE), agg, batch)
